# baseline (speedup 1.0000x reference)
import jax
import jax.numpy as jnp
from jax.experimental import pallas as pl
from jax.experimental.pallas import tpu as pltpu


def _body(x_ref, sym_ref, sc_ref, o_ref):
    sym = sym_ref[...]
    sc = sc_ref[...]
    o_ref[...] = x_ref[...] + sym[None, :, None, :] + sc[None, None, :, :]


def kernel(x, sym_table, sc_table):
    B, T, S, D = x.shape
    BT = 16

    grid = (B, T // BT)
    return pl.pallas_call(
        _body,
        grid=grid,
        in_specs=[
            pl.BlockSpec((1, BT, S, D), lambda b, t: (b, t, 0, 0)),
            pl.BlockSpec((BT, D), lambda b, t: (t, 0)),
            pl.BlockSpec((S, D), lambda b, t: (0, 0)),
        ],
        out_specs=pl.BlockSpec((1, BT, S, D), lambda b, t: (b, t, 0, 0)),
        out_shape=jax.ShapeDtypeStruct(x.shape, x.dtype),
        compiler_params=pltpu.CompilerParams(
            dimension_semantics=("parallel", "parallel"),
        ),
    )(x, sym_table, sc_table)


# speedup vs baseline: 1.0945x; 1.0945x over previous
import jax
import jax.numpy as jnp
from jax.experimental import pallas as pl
from jax.experimental.pallas import tpu as pltpu


def _body(x_ref, sym_ref, sc_ref, o_ref):
    sym = sym_ref[...]
    sc = sc_ref[...]
    o_ref[...] = x_ref[...] + sym[None, :, None, :] + sc[None, None, :, :]


def kernel(x, sym_table, sc_table):
    B, T, S, D = x.shape
    BT = 32

    grid = (B, T // BT)
    return pl.pallas_call(
        _body,
        grid=grid,
        in_specs=[
            pl.BlockSpec((1, BT, S, D), lambda b, t: (b, t, 0, 0)),
            pl.BlockSpec((BT, D), lambda b, t: (t, 0)),
            pl.BlockSpec((S, D), lambda b, t: (0, 0)),
        ],
        out_specs=pl.BlockSpec((1, BT, S, D), lambda b, t: (b, t, 0, 0)),
        out_shape=jax.ShapeDtypeStruct(x.shape, x.dtype),
        compiler_params=pltpu.CompilerParams(
            dimension_semantics=("parallel", "parallel"),
        ),
    )(x, sym_table, sc_table)


# speedup vs baseline: 1.1085x; 1.0128x over previous
import jax
import jax.numpy as jnp
from jax.experimental import pallas as pl
from jax.experimental.pallas import tpu as pltpu


def _body(x_ref, sym_ref, sc_ref, o_ref):
    sym = sym_ref[...]
    sc = sc_ref[...]
    o_ref[...] = x_ref[...] + sym[None, :, None, :] + sc[None, None, :, :]


def kernel(x, sym_table, sc_table):
    B, T, S, D = x.shape
    BT = 64

    grid = (B, T // BT)
    return pl.pallas_call(
        _body,
        grid=grid,
        in_specs=[
            pl.BlockSpec((1, BT, S, D), lambda b, t: (b, t, 0, 0)),
            pl.BlockSpec((BT, D), lambda b, t: (t, 0)),
            pl.BlockSpec((S, D), lambda b, t: (0, 0)),
        ],
        out_specs=pl.BlockSpec((1, BT, S, D), lambda b, t: (b, t, 0, 0)),
        out_shape=jax.ShapeDtypeStruct(x.shape, x.dtype),
        compiler_params=pltpu.CompilerParams(
            dimension_semantics=("parallel", "parallel"),
        ),
    )(x, sym_table, sc_table)
